# baseline (speedup 1.0000x reference)
import functools

import jax
import jax.numpy as jnp
from jax import lax
from jax.experimental import pallas as pl
from jax.experimental.pallas import tpu as pltpu
from jax.experimental.pallas import tpu_sc as plsc

N = 50000
E = 800000
F_IN = 33
H = 64
C = 6
G = 512

NC = 2
NS = 16
LANES = 16

BN = 512
NP = 50176
NBLK = NP // BN
N2 = NP // 2
ROWS_PER_TILE = N2 // NS
ACC_ROWS = N2 + 16
ZROWS = ACC_ROWS // NS

EGROUP = 128
GPT = 392
NGROUPS = GPT * NS
NGROUPS_ALLOC = NGROUPS + 4
EP = NGROUPS_ALLOC * EGROUP

DIN1 = 48
HH = 32

PB_GROUPS = 196
STG = 4
FL = 512
CAPB_CH = 26
CAPB = CAPB_CH * FL
BUCKET_G = CAPB // EGROUP
TOTG = 2 * 32 * BUCKET_G
OUT_E = TOTG * EGROUP
GPT2 = 2 * BUCKET_G


def _make_agg(dp, sch):
  nsc = GPT // sch
  assert nsc * sch == GPT and nsc % 2 == 0
  mesh = plsc.VectorSubcoreMesh(
      core_axis_name="c", subcore_axis_name="s", num_cores=NC, num_subcores=NS)

  @functools.partial(
      pl.kernel,
      out_type=jax.ShapeDtypeStruct((NP, dp), jnp.float32),
      mesh=mesh,
      compiler_params=pltpu.CompilerParams(use_tc_tiling_on_sc=False),
      scratch_types=[
          pltpu.VMEM((2, sch, 2, EGROUP), jnp.int32),
          pltpu.VMEM((2, sch, EGROUP), jnp.int32),
          pltpu.VMEM((2, sch, EGROUP, dp), jnp.float32),
          pltpu.VMEM_SHARED((ACC_ROWS, dp), jnp.float32),
          pltpu.SemaphoreType.DMA,
          pltpu.SemaphoreType.DMA,
          pltpu.SemaphoreType.DMA,
          pltpu.SemaphoreType.DMA,
      ],
  )
  def agg(table_hbm, idx2_hbm, zeros_hbm, out_hbm, idx_v, ldst_v, rows_v,
          acc, semg0, semg1, sems0, sems1):
    cid = lax.axis_index("c")
    sid = lax.axis_index("s")
    base = cid * N2
    semg = (semg0, semg1)
    sems = (sems0, sems1)

    pltpu.sync_copy(zeros_hbm, acc.at[pl.ds(sid * ZROWS, ZROWS)])
    plsc.subcore_barrier()

    def load_idx(c, p):
      grow = sid * GPT + c * sch
      pltpu.sync_copy(idx2_hbm.at[pl.ds(grow, sch)], idx_v.at[p])

    def compute_ldst(p):
      for j in range(sch):
        for k in range(EGROUP // LANES):
          d = idx_v[p, j, 1, pl.ds(k * LANES, LANES)]
          ld = d - base
          ok = (ld >= 0) & (ld < N2)
          ldst_v[p, j, pl.ds(k * LANES, LANES)] = jnp.where(ok, ld, N2)

    def fire_gathers(p):
      for j in range(sch):
        pltpu.async_copy(table_hbm.at[idx_v.at[p, j, 0]], rows_v.at[p, j],
                         semg[p])

    def drain_gathers(p):
      for j in range(sch):
        pltpu.make_async_copy(table_hbm.at[idx_v.at[p, j, 0]],
                              rows_v.at[p, j], semg[p]).wait()

    def fire_scatters(p):
      for j in range(sch):
        pltpu.async_copy(rows_v.at[p, j], acc.at[ldst_v.at[p, j]], sems[p],
                         add=True)

    def drain_scatters(p):
      for j in range(sch):
        pltpu.make_async_copy(rows_v.at[p, j], acc.at[ldst_v.at[p, j]],
                              sems[p]).wait()

    trash16 = jnp.full((LANES,), N2, jnp.int32)
    for j in range(sch):
      for k in range(EGROUP // LANES):
        ldst_v[1, j, pl.ds(k * LANES, LANES)] = trash16
    fire_scatters(1)
    load_idx(0, 0)
    compute_ldst(0)
    fire_gathers(0)

    def body(i, carry):
      for p in range(2):
        c = 2 * i + p
        drain_scatters(1 - p)
        load_idx(c + 1, 1 - p)
        compute_ldst(1 - p)
        drain_gathers(p)
        fire_gathers(1 - p)
        fire_scatters(p)
      return carry

    lax.fori_loop(0, nsc // 2, body, 0)
    drain_scatters(1)
    drain_gathers(0)
    plsc.subcore_barrier()

    pltpu.sync_copy(
        acc.at[pl.ds(sid * ROWS_PER_TILE, ROWS_PER_TILE)],
        out_hbm.at[pl.ds(base + sid * ROWS_PER_TILE, ROWS_PER_TILE)])

  return agg


_PART_MESH = plsc.VectorSubcoreMesh(
    core_axis_name="c", subcore_axis_name="s", num_cores=NC, num_subcores=NS)


@functools.partial(
    pl.kernel,
    out_type=(
        jax.ShapeDtypeStruct((OUT_E,), jnp.int32),
        jax.ShapeDtypeStruct((OUT_E,), jnp.int32),
    ),
    mesh=_PART_MESH,
    compiler_params=pltpu.CompilerParams(
        use_tc_tiling_on_sc=False, needs_layout_passes=False),
    scratch_types=[
        pltpu.VMEM((STG, 2, EGROUP), jnp.int32),
        pltpu.VMEM((FL + EGROUP, ), jnp.int32),
        pltpu.VMEM((FL + EGROUP, ), jnp.int32),
        pltpu.VMEM((FL, ), jnp.int32),
        pltpu.VMEM((FL, ), jnp.int32),
    ],
)
def _partition(idx2_hbm, psrc_hbm, pldst_hbm, stage, b0, b1, sfl, lfl):
  cid = lax.axis_index("c")
  sid = lax.axis_index("s")
  pidx = cid * NS + sid
  prow = pidx * PB_GROUPS
  bufs = (b0, b1)
  iota16 = lax.iota(jnp.int32, 16)
  pad16 = jnp.full((LANES,), N2 << 16, jnp.int32)

  def bucket_base(b):
    return (b * 32 + pidx) * CAPB

  def flush(b, off, nch):
    nch_c = jnp.minimum(nch, CAPB_CH - 1)
    base = bucket_base(b) + nch_c * FL
    for k in range(FL // LANES):
      v = bufs[b][pl.ds(k * LANES, LANES)]
      sfl[pl.ds(k * LANES, LANES)] = v & 0xFFFF
      lfl[pl.ds(k * LANES, LANES)] = lax.shift_right_logical(v, 16)
    pltpu.sync_copy(sfl, psrc_hbm.at[pl.ds(base, FL)])
    pltpu.sync_copy(lfl, pldst_hbm.at[pl.ds(base, FL)])
    for k in range(EGROUP // LANES):
      bufs[b][pl.ds(k * LANES, LANES)] = (
          bufs[b][pl.ds(FL + k * LANES, LANES)])
    return off - FL, nch + 1

  def maybe_flush(b, off, nch):
    return lax.cond(off >= FL, lambda o, n: flush(b, o, n),
                    lambda o, n: (o, n), off, nch)

  def body(i, carry):
    off0, nch0, off1, nch1 = carry
    pltpu.sync_copy(idx2_hbm.at[pl.ds(prow + i * STG, STG)], stage)
    for j in range(STG):
      for k in range(EGROUP // LANES):
        svec = stage[j, 0, pl.ds(k * LANES, LANES)]
        dvec = stage[j, 1, pl.ds(k * LANES, LANES)]
        ok0 = dvec < N2
        ok1 = (dvec >= N2) & (dvec < NP)
        pk0 = svec | (dvec << 16)
        pk1 = svec | ((dvec - N2) << 16)
        _, v0 = plsc.sort_key_val(jnp.where(ok0, 0, 1), pk0)
        b0[pl.ds(off0, LANES)] = v0
        off0 = off0 + jnp.sum(ok0.astype(jnp.int32))
        _, v1 = plsc.sort_key_val(jnp.where(ok1, 0, 1), pk1)
        b1[pl.ds(off1, LANES)] = v1
        off1 = off1 + jnp.sum(ok1.astype(jnp.int32))
      off0, nch0 = maybe_flush(0, off0, nch0)
      off1, nch1 = maybe_flush(1, off1, nch1)
    return off0, nch0, off1, nch1

  zero = jnp.int32(0)
  off0, nch0, off1, nch1 = lax.fori_loop(
      0, PB_GROUPS // STG, body, (zero, zero, zero, zero))

  for b, off, nch in ((0, off0, nch0), (1, off1, nch1)):
    bb = bufs[b]
    for k in range(FL // LANES):
      lane = iota16 + (k * LANES)
      bb[pl.ds(k * LANES, LANES)] = jnp.where(
          lane >= off, pad16, bb[pl.ds(k * LANES, LANES)])
    _, nch = lax.cond(nch < CAPB_CH, lambda o, n, b=b: flush(b, o, n),
                      lambda o, n: (o, n), off, nch)
    for k in range(FL // LANES):
      sfl[pl.ds(k * LANES, LANES)] = jnp.zeros((LANES,), jnp.int32)
      lfl[pl.ds(k * LANES, LANES)] = jnp.full((LANES,), N2, jnp.int32)

    def pad_body(i, carry, b=b):
      base = bucket_base(b) + i * FL
      pltpu.sync_copy(sfl, psrc_hbm.at[pl.ds(base, FL)])
      pltpu.sync_copy(lfl, pldst_hbm.at[pl.ds(base, FL)])
      return carry

    lax.fori_loop(nch, CAPB_CH, pad_body, 0)


def _make_agg_part(dp, sch):
  nsc = GPT2 // sch
  assert nsc * sch == GPT2 and nsc % 2 == 0
  half = nsc // 2
  mesh = plsc.VectorSubcoreMesh(
      core_axis_name="c", subcore_axis_name="s", num_cores=NC, num_subcores=NS)

  @functools.partial(
      pl.kernel,
      out_type=jax.ShapeDtypeStruct((NP, dp), jnp.float32),
      mesh=mesh,
      compiler_params=pltpu.CompilerParams(use_tc_tiling_on_sc=False),
      scratch_types=[
          pltpu.VMEM((2, sch, EGROUP), jnp.int32),
          pltpu.VMEM((2, sch, EGROUP), jnp.int32),
          pltpu.VMEM((2, sch, EGROUP, dp), jnp.float32),
          pltpu.VMEM_SHARED((ACC_ROWS, dp), jnp.float32),
          pltpu.SemaphoreType.DMA,
          pltpu.SemaphoreType.DMA,
          pltpu.SemaphoreType.DMA,
          pltpu.SemaphoreType.DMA,
      ],
  )
  def agg(table_hbm, psrc_hbm, pldst_hbm, zeros_hbm, out_hbm, src_v, ldst_v,
          rows_v, acc, semg0, semg1, sems0, sems1):
    cid = lax.axis_index("c")
    sid = lax.axis_index("s")
    base = cid * N2
    semg = (semg0, semg1)
    sems = (sems0, sems1)
    base_a = (cid * 32 + sid) * BUCKET_G
    base_b = (cid * 32 + NS + sid) * BUCKET_G

    pltpu.sync_copy(zeros_hbm, acc.at[pl.ds(sid * ZROWS, ZROWS)])
    plsc.subcore_barrier()

    def chunk_row(c):
      r = jnp.where(c < half, base_a + c * sch, base_b + (c - half) * sch)
      return jnp.where(c < nsc, r, base_a)

    def load_idx(c, p):
      row = chunk_row(c)
      pltpu.sync_copy(psrc_hbm.at[pl.ds(row, sch)], src_v.at[p])
      pltpu.sync_copy(pldst_hbm.at[pl.ds(row, sch)], ldst_v.at[p])

    def fire_gathers(p):
      for j in range(sch):
        pltpu.async_copy(table_hbm.at[src_v.at[p, j]], rows_v.at[p, j],
                         semg[p])

    def drain_gathers(p):
      for j in range(sch):
        pltpu.make_async_copy(table_hbm.at[src_v.at[p, j]],
                              rows_v.at[p, j], semg[p]).wait()

    def fire_scatters(p):
      for j in range(sch):
        pltpu.async_copy(rows_v.at[p, j], acc.at[ldst_v.at[p, j]], sems[p],
                         add=True)

    def drain_scatters(p):
      for j in range(sch):
        pltpu.make_async_copy(rows_v.at[p, j], acc.at[ldst_v.at[p, j]],
                              sems[p]).wait()

    trash16 = jnp.full((LANES,), N2, jnp.int32)
    for j in range(sch):
      for k in range(EGROUP // LANES):
        ldst_v[1, j, pl.ds(k * LANES, LANES)] = trash16
    fire_scatters(1)
    load_idx(0, 0)
    fire_gathers(0)

    def body(i, carry):
      for p in range(2):
        c = 2 * i + p
        drain_scatters(1 - p)
        load_idx(c + 1, 1 - p)
        drain_gathers(p)
        fire_gathers(1 - p)
        fire_scatters(p)
      return carry

    lax.fori_loop(0, nsc // 2, body, 0)
    drain_scatters(1)
    drain_gathers(0)
    plsc.subcore_barrier()

    pltpu.sync_copy(
        acc.at[pl.ds(sid * ROWS_PER_TILE, ROWS_PER_TILE)],
        out_hbm.at[pl.ds(base + sid * ROWS_PER_TILE, ROWS_PER_TILE)])

  return agg


_agg48 = _make_agg_part(DIN1, 4)
_agg64 = _make_agg_part(H, 1)


def _mlp_body(split_h, x_refs, a_refs, b_ref, wa_ref, ba_ref, g_ref, be_ref,
              m_ref, v_ref, wb_ref, bb_ref, *out_refs):
  z = jnp.concatenate([r[...] for r in x_refs], axis=1) + (
      jnp.concatenate([r[...] for r in a_refs], axis=1))
  h = jnp.dot(z, wa_ref[...], preferred_element_type=jnp.float32) + ba_ref[...]
  scale = g_ref[...] * lax.rsqrt(v_ref[...] + 1e-5)
  h = scale * (h - m_ref[...]) + be_ref[...]
  h = jnp.maximum(h, 0.0)
  h = jnp.dot(h, wb_ref[...], preferred_element_type=jnp.float32) + bb_ref[...]
  h = jnp.maximum(h, 0.0)
  if split_h:
    out_refs[0][...] = h
  p_ref = out_refs[-1]
  bvals = b_ref[0, 0, :]
  ids = lax.broadcasted_iota(jnp.int32, (G, BN), 0)
  onehot = (ids == bvals[None, :]).astype(jnp.float32)
  pp = jnp.dot(onehot, h, preferred_element_type=jnp.float32)

  @pl.when(pl.program_id(0) == 0)
  def _():
    p_ref[...] = pp

  @pl.when(pl.program_id(0) != 0)
  def _():
    p_ref[...] += pp


_FULL = lambda i: (0, 0)
_ROWB = lambda d: pl.BlockSpec((BN, d), lambda i: (i, 0))


def _mlp_specs(din_parts):
  specs = [_ROWB(d) for d in din_parts]
  specs += [_ROWB(d) for d in din_parts]
  specs += [pl.BlockSpec((1, 1, BN), lambda i: (i, 0, 0))]
  din = sum(din_parts)
  specs += [pl.BlockSpec((din, H), _FULL)]
  specs += [pl.BlockSpec((1, H), _FULL)] * 5
  specs += [pl.BlockSpec((H, H), _FULL)]
  specs += [pl.BlockSpec((1, H), _FULL)]
  return specs


def _wrap_mlp(split_h, nparts):
  def body(*refs):
    x_refs = refs[:nparts]
    a_refs = refs[nparts:2 * nparts]
    rest = refs[2 * nparts:]
    _mlp_body(split_h, x_refs, a_refs, *rest)
  return body


_mlp1 = pl.pallas_call(
    _wrap_mlp(True, 1),
    grid=(NBLK,),
    in_specs=_mlp_specs([DIN1]),
    out_specs=[_ROWB(H), pl.BlockSpec((G, H), _FULL)],
    out_shape=[
        jax.ShapeDtypeStruct((NP, H), jnp.float32),
        jax.ShapeDtypeStruct((G, H), jnp.float32),
    ],
)

_mlp2 = pl.pallas_call(
    _wrap_mlp(False, 1),
    grid=(NBLK,),
    in_specs=_mlp_specs([H]),
    out_specs=[pl.BlockSpec((G, H), _FULL)],
    out_shape=[jax.ShapeDtypeStruct((G, H), jnp.float32)],
)


def _final_body(p1_ref, p2_ref, wl_ref, bl_ref, o_ref):
  o = jnp.dot(p1_ref[...], wl_ref[:H, :], preferred_element_type=jnp.float32)
  o += jnp.dot(p2_ref[...], wl_ref[H:, :], preferred_element_type=jnp.float32)
  o_ref[...] = o + bl_ref[...]


_final = pl.pallas_call(
    _final_body,
    out_shape=jax.ShapeDtypeStruct((G, C), jnp.float32),
)


@jax.jit
def _impl(x, edge_index, batch, W1, b1, g1, be1, m1, v1, W2, b2, W3, b3, g2,
          be2, m2, v2, W4, b4, Wl, bl):
  src = edge_index[0]
  dst = edge_index[1]
  srcp = jnp.concatenate([src, jnp.zeros((EP - E,), jnp.int32)]).reshape(
      NGROUPS_ALLOC, EGROUP)
  dstp = jnp.concatenate([dst, jnp.full((EP - E,), NP, jnp.int32)]).reshape(
      NGROUPS_ALLOC, EGROUP)
  idx2 = jnp.stack([srcp, dstp], axis=1)
  xp = jnp.zeros((NP, DIN1), jnp.float32).at[:N, :F_IN].set(x)
  W1p = jnp.zeros((DIN1, H), jnp.float32).at[:F_IN].set(W1)
  batchp = jnp.concatenate(
      [batch, jnp.full((NP - N,), -1, jnp.int32)]).reshape(NBLK, 1, BN)
  z48 = jnp.zeros((ZROWS, DIN1), jnp.float32)
  z64 = jnp.zeros((ZROWS, H), jnp.float32)
  r = lambda a: a.reshape(1, H)

  psrc, pldst = _partition(idx2)
  psrc3 = psrc.reshape(TOTG, EGROUP)
  pldst3 = pldst.reshape(TOTG, EGROUP)
  agg1 = _agg48(xp, psrc3, pldst3, z48)
  h1, p1 = _mlp1(xp, agg1, batchp, W1p, r(b1), r(g1), r(be1), r(m1),
                 r(v1), W2, r(b2))
  agg2 = _agg64(h1, psrc3, pldst3, z64)
  (p2,) = _mlp2(h1, agg2, batchp, W3, r(b3), r(g2), r(be2),
                r(m2), r(v2), W4, r(b4))
  return _final(p1, p2, Wl, bl.reshape(1, C))


def kernel(x, edge_index, batch, W1, b1, g1, be1, m1, v1, W2, b2, W3, b3, g2,
           be2, m2, v2, W4, b4, Wl, bl):
  return _impl(x, edge_index, batch, W1, b1, g1, be1, m1, v1, W2, b2, W3, b3,
               g2, be2, m2, v2, W4, b4, Wl, bl)


# speedup vs baseline: 1.0003x; 1.0003x over previous
import functools

import jax
import jax.numpy as jnp
from jax import lax
from jax.experimental import pallas as pl
from jax.experimental.pallas import tpu as pltpu
from jax.experimental.pallas import tpu_sc as plsc

N = 50000
E = 800000
F_IN = 33
H = 64
C = 6
G = 512

NC = 2
NS = 16
LANES = 16

BN = 512
NP = 50176
NBLK = NP // BN
N2 = NP // 2
ROWS_PER_TILE = N2 // NS
ACC_ROWS = N2 + 16
ZROWS = ACC_ROWS // NS

EGROUP = 128
GPT = 392
NGROUPS = GPT * NS
NGROUPS_ALLOC = NGROUPS + 4
EP = NGROUPS_ALLOC * EGROUP

DIN1 = 48
HH = 32

PB_GROUPS = 196
STG = 4
FL = 512
CAPB_CH = 26
CAPB = CAPB_CH * FL
BUCKET_G = CAPB // EGROUP
TOTG = 2 * 32 * BUCKET_G
OUT_E = TOTG * EGROUP
GPT2 = 2 * BUCKET_G


def _make_agg(dp, sch):
  nsc = GPT // sch
  assert nsc * sch == GPT and nsc % 2 == 0
  mesh = plsc.VectorSubcoreMesh(
      core_axis_name="c", subcore_axis_name="s", num_cores=NC, num_subcores=NS)

  @functools.partial(
      pl.kernel,
      out_type=jax.ShapeDtypeStruct((NP, dp), jnp.float32),
      mesh=mesh,
      compiler_params=pltpu.CompilerParams(use_tc_tiling_on_sc=False),
      scratch_types=[
          pltpu.VMEM((2, sch, 2, EGROUP), jnp.int32),
          pltpu.VMEM((2, sch, EGROUP), jnp.int32),
          pltpu.VMEM((2, sch, EGROUP, dp), jnp.float32),
          pltpu.VMEM_SHARED((ACC_ROWS, dp), jnp.float32),
          pltpu.SemaphoreType.DMA,
          pltpu.SemaphoreType.DMA,
          pltpu.SemaphoreType.DMA,
          pltpu.SemaphoreType.DMA,
      ],
  )
  def agg(table_hbm, idx2_hbm, zeros_hbm, out_hbm, idx_v, ldst_v, rows_v,
          acc, semg0, semg1, sems0, sems1):
    cid = lax.axis_index("c")
    sid = lax.axis_index("s")
    base = cid * N2
    semg = (semg0, semg1)
    sems = (sems0, sems1)

    pltpu.sync_copy(zeros_hbm, acc.at[pl.ds(sid * ZROWS, ZROWS)])
    plsc.subcore_barrier()

    def load_idx(c, p):
      grow = sid * GPT + c * sch
      pltpu.sync_copy(idx2_hbm.at[pl.ds(grow, sch)], idx_v.at[p])

    def compute_ldst(p):
      for j in range(sch):
        for k in range(EGROUP // LANES):
          d = idx_v[p, j, 1, pl.ds(k * LANES, LANES)]
          ld = d - base
          ok = (ld >= 0) & (ld < N2)
          ldst_v[p, j, pl.ds(k * LANES, LANES)] = jnp.where(ok, ld, N2)

    def fire_gathers(p):
      for j in range(sch):
        pltpu.async_copy(table_hbm.at[idx_v.at[p, j, 0]], rows_v.at[p, j],
                         semg[p])

    def drain_gathers(p):
      for j in range(sch):
        pltpu.make_async_copy(table_hbm.at[idx_v.at[p, j, 0]],
                              rows_v.at[p, j], semg[p]).wait()

    def fire_scatters(p):
      for j in range(sch):
        pltpu.async_copy(rows_v.at[p, j], acc.at[ldst_v.at[p, j]], sems[p],
                         add=True)

    def drain_scatters(p):
      for j in range(sch):
        pltpu.make_async_copy(rows_v.at[p, j], acc.at[ldst_v.at[p, j]],
                              sems[p]).wait()

    trash16 = jnp.full((LANES,), N2, jnp.int32)
    for j in range(sch):
      for k in range(EGROUP // LANES):
        ldst_v[1, j, pl.ds(k * LANES, LANES)] = trash16
    fire_scatters(1)
    load_idx(0, 0)
    compute_ldst(0)
    fire_gathers(0)

    def body(i, carry):
      for p in range(2):
        c = 2 * i + p
        drain_scatters(1 - p)
        load_idx(c + 1, 1 - p)
        compute_ldst(1 - p)
        drain_gathers(p)
        fire_gathers(1 - p)
        fire_scatters(p)
      return carry

    lax.fori_loop(0, nsc // 2, body, 0)
    drain_scatters(1)
    drain_gathers(0)
    plsc.subcore_barrier()

    pltpu.sync_copy(
        acc.at[pl.ds(sid * ROWS_PER_TILE, ROWS_PER_TILE)],
        out_hbm.at[pl.ds(base + sid * ROWS_PER_TILE, ROWS_PER_TILE)])

  return agg


_PART_MESH = plsc.VectorSubcoreMesh(
    core_axis_name="c", subcore_axis_name="s", num_cores=NC, num_subcores=NS)


@functools.partial(
    pl.kernel,
    out_type=(
        jax.ShapeDtypeStruct((OUT_E,), jnp.int32),
        jax.ShapeDtypeStruct((OUT_E,), jnp.int32),
    ),
    mesh=_PART_MESH,
    compiler_params=pltpu.CompilerParams(
        use_tc_tiling_on_sc=False, needs_layout_passes=False),
    scratch_types=[
        pltpu.VMEM((STG, 2, EGROUP), jnp.int32),
        pltpu.VMEM((FL + EGROUP, ), jnp.int32),
        pltpu.VMEM((FL + EGROUP, ), jnp.int32),
        pltpu.VMEM((FL, ), jnp.int32),
        pltpu.VMEM((FL, ), jnp.int32),
    ],
)
def _partition(idx2_hbm, psrc_hbm, pldst_hbm, stage, b0, b1, sfl, lfl):
  cid = lax.axis_index("c")
  sid = lax.axis_index("s")
  pidx = cid * NS + sid
  prow = pidx * PB_GROUPS
  bufs = (b0, b1)
  iota16 = lax.iota(jnp.int32, 16)
  pad16 = jnp.full((LANES,), N2 << 16, jnp.int32)

  def bucket_base(b):
    return ((b * NS + sid) * 2 + cid) * CAPB

  def flush(b, off, nch):
    nch_c = jnp.minimum(nch, CAPB_CH - 1)
    base = bucket_base(b) + nch_c * FL
    for k in range(FL // LANES):
      v = bufs[b][pl.ds(k * LANES, LANES)]
      sfl[pl.ds(k * LANES, LANES)] = v & 0xFFFF
      lfl[pl.ds(k * LANES, LANES)] = lax.shift_right_logical(v, 16)
    pltpu.sync_copy(sfl, psrc_hbm.at[pl.ds(base, FL)])
    pltpu.sync_copy(lfl, pldst_hbm.at[pl.ds(base, FL)])
    for k in range(EGROUP // LANES):
      bufs[b][pl.ds(k * LANES, LANES)] = (
          bufs[b][pl.ds(FL + k * LANES, LANES)])
    return off - FL, nch + 1

  def maybe_flush(b, off, nch):
    return lax.cond(off >= FL, lambda o, n: flush(b, o, n),
                    lambda o, n: (o, n), off, nch)

  def body(i, carry):
    off0, nch0, off1, nch1 = carry
    pltpu.sync_copy(idx2_hbm.at[pl.ds(prow + i * STG, STG)], stage)
    for j in range(STG):
      for k in range(EGROUP // LANES):
        svec = stage[j, 0, pl.ds(k * LANES, LANES)]
        dvec = stage[j, 1, pl.ds(k * LANES, LANES)]
        ok0 = dvec < N2
        ok1 = (dvec >= N2) & (dvec < NP)
        pk0 = svec | (dvec << 16)
        pk1 = svec | ((dvec - N2) << 16)
        _, v0 = plsc.sort_key_val(jnp.where(ok0, 0, 1), pk0)
        b0[pl.ds(off0, LANES)] = v0
        off0 = off0 + jnp.sum(ok0.astype(jnp.int32))
        _, v1 = plsc.sort_key_val(jnp.where(ok1, 0, 1), pk1)
        b1[pl.ds(off1, LANES)] = v1
        off1 = off1 + jnp.sum(ok1.astype(jnp.int32))
      off0, nch0 = maybe_flush(0, off0, nch0)
      off1, nch1 = maybe_flush(1, off1, nch1)
    return off0, nch0, off1, nch1

  zero = jnp.int32(0)
  off0, nch0, off1, nch1 = lax.fori_loop(
      0, PB_GROUPS // STG, body, (zero, zero, zero, zero))

  for b, off, nch in ((0, off0, nch0), (1, off1, nch1)):
    bb = bufs[b]
    for k in range(FL // LANES):
      lane = iota16 + (k * LANES)
      bb[pl.ds(k * LANES, LANES)] = jnp.where(
          lane >= off, pad16, bb[pl.ds(k * LANES, LANES)])
    _, nch = lax.cond(nch < CAPB_CH, lambda o, n, b=b: flush(b, o, n),
                      lambda o, n: (o, n), off, nch)
    for k in range(FL // LANES):
      sfl[pl.ds(k * LANES, LANES)] = jnp.zeros((LANES,), jnp.int32)
      lfl[pl.ds(k * LANES, LANES)] = jnp.full((LANES,), N2, jnp.int32)

    def pad_body(i, carry, b=b):
      base = bucket_base(b) + i * FL
      pltpu.sync_copy(sfl, psrc_hbm.at[pl.ds(base, FL)])
      pltpu.sync_copy(lfl, pldst_hbm.at[pl.ds(base, FL)])
      return carry

    lax.fori_loop(nch, CAPB_CH, pad_body, 0)


def _make_agg_part(dp, sch):
  nsc = GPT2 // sch
  assert nsc * sch == GPT2 and nsc % 2 == 0
  half = nsc // 2
  mesh = plsc.VectorSubcoreMesh(
      core_axis_name="c", subcore_axis_name="s", num_cores=NC, num_subcores=NS)

  @functools.partial(
      pl.kernel,
      out_type=jax.ShapeDtypeStruct((NP, dp), jnp.float32),
      mesh=mesh,
      compiler_params=pltpu.CompilerParams(use_tc_tiling_on_sc=False),
      scratch_types=[
          pltpu.VMEM((2, sch, EGROUP), jnp.int32),
          pltpu.VMEM((2, sch, EGROUP), jnp.int32),
          pltpu.VMEM((2, sch, EGROUP, dp), jnp.float32),
          pltpu.VMEM_SHARED((ACC_ROWS, dp), jnp.float32),
          pltpu.SemaphoreType.DMA,
          pltpu.SemaphoreType.DMA,
          pltpu.SemaphoreType.DMA,
          pltpu.SemaphoreType.DMA,
      ],
  )
  def agg(table_hbm, psrc_hbm, pldst_hbm, zeros_hbm, out_hbm, src_v, ldst_v,
          rows_v, acc, semg0, semg1, sems0, sems1):
    cid = lax.axis_index("c")
    sid = lax.axis_index("s")
    base = cid * N2
    semg = (semg0, semg1)
    sems = (sems0, sems1)
    base0 = (cid * NS + sid) * 2 * BUCKET_G

    pltpu.sync_copy(zeros_hbm, acc.at[pl.ds(sid * ZROWS, ZROWS)])
    plsc.subcore_barrier()

    def chunk_row(c):
      return jnp.where(c < nsc, base0 + c * sch, base0)

    def load_idx(c, p):
      row = chunk_row(c)
      pltpu.sync_copy(psrc_hbm.at[pl.ds(row, sch)], src_v.at[p])
      pltpu.sync_copy(pldst_hbm.at[pl.ds(row, sch)], ldst_v.at[p])

    def fire_gathers(p):
      for j in range(sch):
        pltpu.async_copy(table_hbm.at[src_v.at[p, j]], rows_v.at[p, j],
                         semg[p])

    def drain_gathers(p):
      for j in range(sch):
        pltpu.make_async_copy(table_hbm.at[src_v.at[p, j]],
                              rows_v.at[p, j], semg[p]).wait()

    def fire_scatters(p):
      for j in range(sch):
        pltpu.async_copy(rows_v.at[p, j], acc.at[ldst_v.at[p, j]], sems[p],
                         add=True)

    def drain_scatters(p):
      for j in range(sch):
        pltpu.make_async_copy(rows_v.at[p, j], acc.at[ldst_v.at[p, j]],
                              sems[p]).wait()

    trash16 = jnp.full((LANES,), N2, jnp.int32)
    for j in range(sch):
      for k in range(EGROUP // LANES):
        ldst_v[1, j, pl.ds(k * LANES, LANES)] = trash16
    fire_scatters(1)
    load_idx(0, 0)
    fire_gathers(0)

    def body(i, carry):
      for p in range(2):
        c = 2 * i + p
        drain_scatters(1 - p)
        load_idx(c + 1, 1 - p)
        drain_gathers(p)
        fire_gathers(1 - p)
        fire_scatters(p)
      return carry

    lax.fori_loop(0, nsc // 2, body, 0)
    drain_scatters(1)
    drain_gathers(0)
    plsc.subcore_barrier()

    pltpu.sync_copy(
        acc.at[pl.ds(sid * ROWS_PER_TILE, ROWS_PER_TILE)],
        out_hbm.at[pl.ds(base + sid * ROWS_PER_TILE, ROWS_PER_TILE)])

  return agg


_agg48 = _make_agg_part(DIN1, 4)
_agg64 = _make_agg_part(H, 1)


def _mlp_body(split_h, x_refs, a_refs, b_ref, wa_ref, ba_ref, g_ref, be_ref,
              m_ref, v_ref, wb_ref, bb_ref, *out_refs):
  z = jnp.concatenate([r[...] for r in x_refs], axis=1) + (
      jnp.concatenate([r[...] for r in a_refs], axis=1))
  h = jnp.dot(z, wa_ref[...], preferred_element_type=jnp.float32) + ba_ref[...]
  scale = g_ref[...] * lax.rsqrt(v_ref[...] + 1e-5)
  h = scale * (h - m_ref[...]) + be_ref[...]
  h = jnp.maximum(h, 0.0)
  h = jnp.dot(h, wb_ref[...], preferred_element_type=jnp.float32) + bb_ref[...]
  h = jnp.maximum(h, 0.0)
  if split_h:
    out_refs[0][...] = h
  p_ref = out_refs[-1]
  bvals = b_ref[0, 0, :]
  ids = lax.broadcasted_iota(jnp.int32, (G, BN), 0)
  onehot = (ids == bvals[None, :]).astype(jnp.float32)
  pp = jnp.dot(onehot, h, preferred_element_type=jnp.float32)

  @pl.when(pl.program_id(0) == 0)
  def _():
    p_ref[...] = pp

  @pl.when(pl.program_id(0) != 0)
  def _():
    p_ref[...] += pp


_FULL = lambda i: (0, 0)
_ROWB = lambda d: pl.BlockSpec((BN, d), lambda i: (i, 0))


def _mlp_specs(din_parts):
  specs = [_ROWB(d) for d in din_parts]
  specs += [_ROWB(d) for d in din_parts]
  specs += [pl.BlockSpec((1, 1, BN), lambda i: (i, 0, 0))]
  din = sum(din_parts)
  specs += [pl.BlockSpec((din, H), _FULL)]
  specs += [pl.BlockSpec((1, H), _FULL)] * 5
  specs += [pl.BlockSpec((H, H), _FULL)]
  specs += [pl.BlockSpec((1, H), _FULL)]
  return specs


def _wrap_mlp(split_h, nparts):
  def body(*refs):
    x_refs = refs[:nparts]
    a_refs = refs[nparts:2 * nparts]
    rest = refs[2 * nparts:]
    _mlp_body(split_h, x_refs, a_refs, *rest)
  return body


_mlp1 = pl.pallas_call(
    _wrap_mlp(True, 1),
    grid=(NBLK,),
    in_specs=_mlp_specs([DIN1]),
    out_specs=[_ROWB(H), pl.BlockSpec((G, H), _FULL)],
    out_shape=[
        jax.ShapeDtypeStruct((NP, H), jnp.float32),
        jax.ShapeDtypeStruct((G, H), jnp.float32),
    ],
)

_mlp2 = pl.pallas_call(
    _wrap_mlp(False, 1),
    grid=(NBLK,),
    in_specs=_mlp_specs([H]),
    out_specs=[pl.BlockSpec((G, H), _FULL)],
    out_shape=[jax.ShapeDtypeStruct((G, H), jnp.float32)],
)


def _final_body(p1_ref, p2_ref, wl_ref, bl_ref, o_ref):
  o = jnp.dot(p1_ref[...], wl_ref[:H, :], preferred_element_type=jnp.float32)
  o += jnp.dot(p2_ref[...], wl_ref[H:, :], preferred_element_type=jnp.float32)
  o_ref[...] = o + bl_ref[...]


_final = pl.pallas_call(
    _final_body,
    out_shape=jax.ShapeDtypeStruct((G, C), jnp.float32),
)


@jax.jit
def _impl(x, edge_index, batch, W1, b1, g1, be1, m1, v1, W2, b2, W3, b3, g2,
          be2, m2, v2, W4, b4, Wl, bl):
  src = edge_index[0]
  dst = edge_index[1]
  srcp = jnp.concatenate([src, jnp.zeros((EP - E,), jnp.int32)]).reshape(
      NGROUPS_ALLOC, EGROUP)
  dstp = jnp.concatenate([dst, jnp.full((EP - E,), NP, jnp.int32)]).reshape(
      NGROUPS_ALLOC, EGROUP)
  idx2 = jnp.stack([srcp, dstp], axis=1)
  xp = jnp.zeros((NP, DIN1), jnp.float32).at[:N, :F_IN].set(x)
  W1p = jnp.zeros((DIN1, H), jnp.float32).at[:F_IN].set(W1)
  batchp = jnp.concatenate(
      [batch, jnp.full((NP - N,), -1, jnp.int32)]).reshape(NBLK, 1, BN)
  z48 = jnp.zeros((ZROWS, DIN1), jnp.float32)
  z64 = jnp.zeros((ZROWS, H), jnp.float32)
  r = lambda a: a.reshape(1, H)

  psrc, pldst = _partition(idx2)
  psrc3 = psrc.reshape(TOTG, EGROUP)
  pldst3 = pldst.reshape(TOTG, EGROUP)
  agg1 = _agg48(xp, psrc3, pldst3, z48)
  h1, p1 = _mlp1(xp, agg1, batchp, W1p, r(b1), r(g1), r(be1), r(m1),
                 r(v1), W2, r(b2))
  agg2 = _agg64(h1, psrc3, pldst3, z64)
  (p2,) = _mlp2(h1, agg2, batchp, W3, r(b3), r(g2), r(be2),
                r(m2), r(v2), W4, r(b4))
  return _final(p1, p2, Wl, bl.reshape(1, C))


def kernel(x, edge_index, batch, W1, b1, g1, be1, m1, v1, W2, b2, W3, b3, g2,
           be2, m2, v2, W4, b4, Wl, bl):
  return _impl(x, edge_index, batch, W1, b1, g1, be1, m1, v1, W2, b2, W3, b3,
               g2, be2, m2, v2, W4, b4, Wl, bl)


# speedup vs baseline: 1.8279x; 1.8274x over previous
import functools

import jax
import jax.numpy as jnp
from jax import lax
from jax.experimental import pallas as pl
from jax.experimental.pallas import tpu as pltpu
from jax.experimental.pallas import tpu_sc as plsc

N = 50000
E = 800000
F_IN = 33
H = 64
C = 6
G = 512

NC = 2
NS = 16
LANES = 16

BN = 512
NP = 50176
NBLK = NP // BN
N2 = NP // 2
ROWS_PER_TILE = N2 // NS
ACC_ROWS = N2 + 16
ZROWS = ACC_ROWS // NS

EGROUP = 128
GPT = 392
NGROUPS = GPT * NS
NGROUPS_ALLOC = NGROUPS + 4
EP = NGROUPS_ALLOC * EGROUP

DIN1 = 48
HH = 32

PB_GROUPS = 196
STG = 4
FL = 512
CAPB_CH = 26
CAPB = CAPB_CH * FL
BUCKET_G = CAPB // EGROUP
TOTG = 2 * 32 * BUCKET_G
OUT_E = TOTG * EGROUP
GPT2 = 2 * BUCKET_G


def _make_agg(dp, sch, dt):
  nsc = GPT // sch
  assert nsc * sch == GPT and nsc % 2 == 0
  mesh = plsc.VectorSubcoreMesh(
      core_axis_name="c", subcore_axis_name="s", num_cores=NC, num_subcores=NS)

  @functools.partial(
      pl.kernel,
      out_type=jax.ShapeDtypeStruct((NP, dp), dt),
      mesh=mesh,
      compiler_params=pltpu.CompilerParams(use_tc_tiling_on_sc=False),
      scratch_types=[
          pltpu.VMEM((2, sch, 2, EGROUP), jnp.int32),
          pltpu.VMEM((2, sch, EGROUP), jnp.int32),
          pltpu.VMEM((2, sch, EGROUP, dp), dt),
          pltpu.VMEM_SHARED((ACC_ROWS, dp), dt),
          pltpu.SemaphoreType.DMA,
          pltpu.SemaphoreType.DMA,
          pltpu.SemaphoreType.DMA,
          pltpu.SemaphoreType.DMA,
      ],
  )
  def agg(table_hbm, idx2_hbm, zeros_hbm, out_hbm, idx_v, ldst_v, rows_v,
          acc, semg0, semg1, sems0, sems1):
    cid = lax.axis_index("c")
    sid = lax.axis_index("s")
    base = cid * N2
    semg = (semg0, semg1)
    sems = (sems0, sems1)

    pltpu.sync_copy(zeros_hbm, acc.at[pl.ds(sid * ZROWS, ZROWS)])
    plsc.subcore_barrier()

    def load_idx(c, p):
      grow = sid * GPT + c * sch
      pltpu.sync_copy(idx2_hbm.at[pl.ds(grow, sch)], idx_v.at[p])

    def compute_ldst(p):
      for j in range(sch):
        for k in range(EGROUP // LANES):
          d = idx_v[p, j, 1, pl.ds(k * LANES, LANES)]
          ld = d - base
          ok = (ld >= 0) & (ld < N2)
          ldst_v[p, j, pl.ds(k * LANES, LANES)] = jnp.where(ok, ld, N2)

    def fire_gathers(p):
      for j in range(sch):
        pltpu.async_copy(table_hbm.at[idx_v.at[p, j, 0]], rows_v.at[p, j],
                         semg[p])

    def drain_gathers(p):
      for j in range(sch):
        pltpu.make_async_copy(table_hbm.at[idx_v.at[p, j, 0]],
                              rows_v.at[p, j], semg[p]).wait()

    def fire_scatters(p):
      for j in range(sch):
        pltpu.async_copy(rows_v.at[p, j], acc.at[ldst_v.at[p, j]], sems[p],
                         add=True)

    def drain_scatters(p):
      for j in range(sch):
        pltpu.make_async_copy(rows_v.at[p, j], acc.at[ldst_v.at[p, j]],
                              sems[p]).wait()

    trash16 = jnp.full((LANES,), N2, jnp.int32)
    for j in range(sch):
      for k in range(EGROUP // LANES):
        ldst_v[1, j, pl.ds(k * LANES, LANES)] = trash16
    fire_scatters(1)
    load_idx(0, 0)
    compute_ldst(0)
    fire_gathers(0)

    def body(i, carry):
      for p in range(2):
        c = 2 * i + p
        drain_scatters(1 - p)
        load_idx(c + 1, 1 - p)
        compute_ldst(1 - p)
        drain_gathers(p)
        fire_gathers(1 - p)
        fire_scatters(p)
      return carry

    lax.fori_loop(0, nsc // 2, body, 0)
    drain_scatters(1)
    drain_gathers(0)
    plsc.subcore_barrier()

    pltpu.sync_copy(
        acc.at[pl.ds(sid * ROWS_PER_TILE, ROWS_PER_TILE)],
        out_hbm.at[pl.ds(base + sid * ROWS_PER_TILE, ROWS_PER_TILE)])

  return agg


_agg48 = _make_agg(DIN1, 4, jnp.float32)
_agg64 = _make_agg(H, 2, jnp.bfloat16)


def _mlp_body(split_h, x_refs, a_refs, b_ref, wa_ref, ba_ref, g_ref, be_ref,
              m_ref, v_ref, wb_ref, bb_ref, *out_refs):
  z = jnp.concatenate([r[...] for r in x_refs], axis=1) + (
      jnp.concatenate([r[...] for r in a_refs], axis=1).astype(jnp.float32))
  h = jnp.dot(z, wa_ref[...], preferred_element_type=jnp.float32) + ba_ref[...]
  scale = g_ref[...] * lax.rsqrt(v_ref[...] + 1e-5)
  h = scale * (h - m_ref[...]) + be_ref[...]
  h = jnp.maximum(h, 0.0)
  h = jnp.dot(h, wb_ref[...], preferred_element_type=jnp.float32) + bb_ref[...]
  h = jnp.maximum(h, 0.0)
  if split_h:
    out_refs[0][...] = h
    out_refs[1][...] = h.astype(jnp.bfloat16)
  p_ref = out_refs[-1]
  bvals = b_ref[0, 0, :]
  ids = lax.broadcasted_iota(jnp.int32, (G, BN), 0)
  onehot = (ids == bvals[None, :]).astype(jnp.float32)
  pp = jnp.dot(onehot, h, preferred_element_type=jnp.float32)

  @pl.when(pl.program_id(0) == 0)
  def _():
    p_ref[...] = pp

  @pl.when(pl.program_id(0) != 0)
  def _():
    p_ref[...] += pp


_FULL = lambda i: (0, 0)
_ROWB = lambda d: pl.BlockSpec((BN, d), lambda i: (i, 0))


def _mlp_specs(din_parts):
  specs = [_ROWB(d) for d in din_parts]
  specs += [_ROWB(d) for d in din_parts]
  specs += [pl.BlockSpec((1, 1, BN), lambda i: (i, 0, 0))]
  din = sum(din_parts)
  specs += [pl.BlockSpec((din, H), _FULL)]
  specs += [pl.BlockSpec((1, H), _FULL)] * 5
  specs += [pl.BlockSpec((H, H), _FULL)]
  specs += [pl.BlockSpec((1, H), _FULL)]
  return specs


def _wrap_mlp(split_h, nparts):
  def body(*refs):
    x_refs = refs[:nparts]
    a_refs = refs[nparts:2 * nparts]
    rest = refs[2 * nparts:]
    _mlp_body(split_h, x_refs, a_refs, *rest)
  return body


_mlp1 = pl.pallas_call(
    _wrap_mlp(True, 1),
    grid=(NBLK,),
    in_specs=_mlp_specs([DIN1]),
    out_specs=[_ROWB(H), _ROWB(H), pl.BlockSpec((G, H), _FULL)],
    out_shape=[
        jax.ShapeDtypeStruct((NP, H), jnp.float32),
        jax.ShapeDtypeStruct((NP, H), jnp.bfloat16),
        jax.ShapeDtypeStruct((G, H), jnp.float32),
    ],
)

_mlp2 = pl.pallas_call(
    _wrap_mlp(False, 1),
    grid=(NBLK,),
    in_specs=_mlp_specs([H]),
    out_specs=[pl.BlockSpec((G, H), _FULL)],
    out_shape=[jax.ShapeDtypeStruct((G, H), jnp.float32)],
)


def _final_body(p1_ref, p2_ref, wl_ref, bl_ref, o_ref):
  o = jnp.dot(p1_ref[...], wl_ref[:H, :], preferred_element_type=jnp.float32)
  o += jnp.dot(p2_ref[...], wl_ref[H:, :], preferred_element_type=jnp.float32)
  o_ref[...] = o + bl_ref[...]


_final = pl.pallas_call(
    _final_body,
    out_shape=jax.ShapeDtypeStruct((G, C), jnp.float32),
)


@jax.jit
def _impl(x, edge_index, batch, W1, b1, g1, be1, m1, v1, W2, b2, W3, b3, g2,
          be2, m2, v2, W4, b4, Wl, bl):
  src = edge_index[0]
  dst = edge_index[1]
  srcp = jnp.concatenate([src, jnp.zeros((EP - E,), jnp.int32)]).reshape(
      NGROUPS_ALLOC, EGROUP)
  dstp = jnp.concatenate([dst, jnp.full((EP - E,), NP, jnp.int32)]).reshape(
      NGROUPS_ALLOC, EGROUP)
  idx2 = jnp.stack([srcp, dstp], axis=1)
  xp = jnp.zeros((NP, DIN1), jnp.float32).at[:N, :F_IN].set(x)
  W1p = jnp.zeros((DIN1, H), jnp.float32).at[:F_IN].set(W1)
  batchp = jnp.concatenate(
      [batch, jnp.full((NP - N,), -1, jnp.int32)]).reshape(NBLK, 1, BN)
  z48 = jnp.zeros((ZROWS, DIN1), jnp.float32)
  z64 = jnp.zeros((ZROWS, H), jnp.bfloat16)
  r = lambda a: a.reshape(1, H)

  agg1 = _agg48(xp, idx2, z48)
  h1, h1bf, p1 = _mlp1(xp, agg1, batchp, W1p, r(b1), r(g1), r(be1), r(m1),
                       r(v1), W2, r(b2))
  agg2 = _agg64(h1bf, idx2, z64)
  (p2,) = _mlp2(h1, agg2, batchp, W3, r(b3), r(g2), r(be2),
                r(m2), r(v2), W4, r(b4))
  return _final(p1, p2, Wl, bl.reshape(1, C))


def kernel(x, edge_index, batch, W1, b1, g1, be1, m1, v1, W2, b2, W3, b3, g2,
           be2, m2, v2, W4, b4, Wl, bl):
  return _impl(x, edge_index, batch, W1, b1, g1, be1, m1, v1, W2, b2, W3, b3,
               g2, be2, m2, v2, W4, b4, Wl, bl)


# speedup vs baseline: 2.0727x; 1.1339x over previous
import functools

import jax
import jax.numpy as jnp
from jax import lax
from jax.experimental import pallas as pl
from jax.experimental.pallas import tpu as pltpu
from jax.experimental.pallas import tpu_sc as plsc

N = 50000
E = 800000
F_IN = 33
H = 64
C = 6
G = 512

NC = 2
NS = 16
LANES = 16

BN = 512
NP = 50176
NBLK = NP // BN
N2 = NP // 2
ROWS_PER_TILE = N2 // NS
ACC_ROWS = N2 + 16
ZROWS = ACC_ROWS // NS

EGROUP = 128
GPT = 392
NGROUPS = GPT * NS
NGROUPS_ALLOC = NGROUPS + 4
EP = NGROUPS_ALLOC * EGROUP

DIN1 = 64
HH = 32

PB_GROUPS = 196
STG = 4
FL = 512
CAPB_CH = 26
CAPB = CAPB_CH * FL
BUCKET_G = CAPB // EGROUP
TOTG = 2 * 32 * BUCKET_G
OUT_E = TOTG * EGROUP
GPT2 = 2 * BUCKET_G


def _make_agg(dp, sch, dt):
  nsc = GPT // sch
  assert nsc * sch == GPT and nsc % 2 == 0
  mesh = plsc.VectorSubcoreMesh(
      core_axis_name="c", subcore_axis_name="s", num_cores=NC, num_subcores=NS)

  @functools.partial(
      pl.kernel,
      out_type=jax.ShapeDtypeStruct((NP, dp), dt),
      mesh=mesh,
      compiler_params=pltpu.CompilerParams(use_tc_tiling_on_sc=False),
      scratch_types=[
          pltpu.VMEM((2, sch, 2, EGROUP), jnp.int32),
          pltpu.VMEM((2, sch, EGROUP), jnp.int32),
          pltpu.VMEM((2, sch, EGROUP, dp), dt),
          pltpu.VMEM_SHARED((ACC_ROWS, dp), dt),
          pltpu.SemaphoreType.DMA,
          pltpu.SemaphoreType.DMA,
          pltpu.SemaphoreType.DMA,
          pltpu.SemaphoreType.DMA,
      ],
  )
  def agg(table_hbm, idx2_hbm, zeros_hbm, out_hbm, idx_v, ldst_v, rows_v,
          acc, semg0, semg1, sems0, sems1):
    cid = lax.axis_index("c")
    sid = lax.axis_index("s")
    base = cid * N2
    semg = (semg0, semg1)
    sems = (sems0, sems1)

    pltpu.sync_copy(zeros_hbm, acc.at[pl.ds(sid * ZROWS, ZROWS)])
    plsc.subcore_barrier()

    def load_idx(c, p):
      grow = sid * GPT + c * sch
      pltpu.sync_copy(idx2_hbm.at[pl.ds(grow, sch)], idx_v.at[p])

    def compute_ldst(p):
      for j in range(sch):
        for k in range(EGROUP // LANES):
          d = idx_v[p, j, 1, pl.ds(k * LANES, LANES)]
          ld = d - base
          ok = (ld >= 0) & (ld < N2)
          ldst_v[p, j, pl.ds(k * LANES, LANES)] = jnp.where(ok, ld, N2)

    def fire_gathers(p):
      for j in range(sch):
        pltpu.async_copy(table_hbm.at[idx_v.at[p, j, 0]], rows_v.at[p, j],
                         semg[p])

    def drain_gathers(p):
      for j in range(sch):
        pltpu.make_async_copy(table_hbm.at[idx_v.at[p, j, 0]],
                              rows_v.at[p, j], semg[p]).wait()

    def fire_scatters(p):
      for j in range(sch):
        pltpu.async_copy(rows_v.at[p, j], acc.at[ldst_v.at[p, j]], sems[p],
                         add=True)

    def drain_scatters(p):
      for j in range(sch):
        pltpu.make_async_copy(rows_v.at[p, j], acc.at[ldst_v.at[p, j]],
                              sems[p]).wait()

    trash16 = jnp.full((LANES,), N2, jnp.int32)
    for j in range(sch):
      for k in range(EGROUP // LANES):
        ldst_v[1, j, pl.ds(k * LANES, LANES)] = trash16
    fire_scatters(1)
    load_idx(0, 0)
    compute_ldst(0)
    fire_gathers(0)

    def body(i, carry):
      for p in range(2):
        c = 2 * i + p
        drain_scatters(1 - p)
        load_idx(c + 1, 1 - p)
        compute_ldst(1 - p)
        drain_gathers(p)
        fire_gathers(1 - p)
        fire_scatters(p)
      return carry

    lax.fori_loop(0, nsc // 2, body, 0)
    drain_scatters(1)
    drain_gathers(0)
    plsc.subcore_barrier()

    pltpu.sync_copy(
        acc.at[pl.ds(sid * ROWS_PER_TILE, ROWS_PER_TILE)],
        out_hbm.at[pl.ds(base + sid * ROWS_PER_TILE, ROWS_PER_TILE)])

  return agg


_agg64 = _make_agg(H, 2, jnp.bfloat16)


def _mlp_body(split_h, x_refs, a_refs, b_ref, wa_ref, ba_ref, g_ref, be_ref,
              m_ref, v_ref, wb_ref, bb_ref, *out_refs):
  z = jnp.concatenate([r[...] for r in x_refs], axis=1) + (
      jnp.concatenate([r[...] for r in a_refs], axis=1).astype(jnp.float32))
  h = jnp.dot(z, wa_ref[...], preferred_element_type=jnp.float32) + ba_ref[...]
  scale = g_ref[...] * lax.rsqrt(v_ref[...] + 1e-5)
  h = scale * (h - m_ref[...]) + be_ref[...]
  h = jnp.maximum(h, 0.0)
  h = jnp.dot(h, wb_ref[...], preferred_element_type=jnp.float32) + bb_ref[...]
  h = jnp.maximum(h, 0.0)
  if split_h:
    out_refs[0][...] = h
    out_refs[1][...] = h.astype(jnp.bfloat16)
  p_ref = out_refs[-1]
  bvals = b_ref[0, 0, :]
  ids = lax.broadcasted_iota(jnp.int32, (G, BN), 0)
  onehot = (ids == bvals[None, :]).astype(jnp.float32)
  pp = jnp.dot(onehot, h, preferred_element_type=jnp.float32)

  @pl.when(pl.program_id(0) == 0)
  def _():
    p_ref[...] = pp

  @pl.when(pl.program_id(0) != 0)
  def _():
    p_ref[...] += pp


_FULL = lambda i: (0, 0)
_ROWB = lambda d: pl.BlockSpec((BN, d), lambda i: (i, 0))


def _mlp_specs(din_parts):
  specs = [_ROWB(d) for d in din_parts]
  specs += [_ROWB(d) for d in din_parts]
  specs += [pl.BlockSpec((1, 1, BN), lambda i: (i, 0, 0))]
  din = sum(din_parts)
  specs += [pl.BlockSpec((din, H), _FULL)]
  specs += [pl.BlockSpec((1, H), _FULL)] * 5
  specs += [pl.BlockSpec((H, H), _FULL)]
  specs += [pl.BlockSpec((1, H), _FULL)]
  return specs


def _wrap_mlp(split_h, nparts):
  def body(*refs):
    x_refs = refs[:nparts]
    a_refs = refs[nparts:2 * nparts]
    rest = refs[2 * nparts:]
    _mlp_body(split_h, x_refs, a_refs, *rest)
  return body


_mlp1 = pl.pallas_call(
    _wrap_mlp(True, 1),
    grid=(NBLK,),
    in_specs=_mlp_specs([DIN1]),
    out_specs=[_ROWB(H), _ROWB(H), pl.BlockSpec((G, H), _FULL)],
    out_shape=[
        jax.ShapeDtypeStruct((NP, H), jnp.float32),
        jax.ShapeDtypeStruct((NP, H), jnp.bfloat16),
        jax.ShapeDtypeStruct((G, H), jnp.float32),
    ],
)

_mlp2 = pl.pallas_call(
    _wrap_mlp(False, 1),
    grid=(NBLK,),
    in_specs=_mlp_specs([H]),
    out_specs=[pl.BlockSpec((G, H), _FULL)],
    out_shape=[jax.ShapeDtypeStruct((G, H), jnp.float32)],
)


def _final_body(p1_ref, p2_ref, wl_ref, bl_ref, o_ref):
  o = jnp.dot(p1_ref[...], wl_ref[:H, :], preferred_element_type=jnp.float32)
  o += jnp.dot(p2_ref[...], wl_ref[H:, :], preferred_element_type=jnp.float32)
  o_ref[...] = o + bl_ref[...]


_final = pl.pallas_call(
    _final_body,
    out_shape=jax.ShapeDtypeStruct((G, C), jnp.float32),
)


@jax.jit
def _impl(x, edge_index, batch, W1, b1, g1, be1, m1, v1, W2, b2, W3, b3, g2,
          be2, m2, v2, W4, b4, Wl, bl):
  src = edge_index[0]
  dst = edge_index[1]
  srcp = jnp.concatenate([src, jnp.zeros((EP - E,), jnp.int32)]).reshape(
      NGROUPS_ALLOC, EGROUP)
  dstp = jnp.concatenate([dst, jnp.full((EP - E,), NP, jnp.int32)]).reshape(
      NGROUPS_ALLOC, EGROUP)
  idx2 = jnp.stack([srcp, dstp], axis=1)
  xp = jnp.zeros((NP, DIN1), jnp.float32).at[:N, :F_IN].set(x)
  W1p = jnp.zeros((DIN1, H), jnp.float32).at[:F_IN].set(W1)
  batchp = jnp.concatenate(
      [batch, jnp.full((NP - N,), -1, jnp.int32)]).reshape(NBLK, 1, BN)
  z64 = jnp.zeros((ZROWS, H), jnp.bfloat16)
  r = lambda a: a.reshape(1, H)

  xpbf = xp.astype(jnp.bfloat16)
  agg1 = _agg64(xpbf, idx2, z64)
  h1, h1bf, p1 = _mlp1(xp, agg1, batchp, W1p, r(b1), r(g1), r(be1), r(m1),
                       r(v1), W2, r(b2))
  agg2 = _agg64(h1bf, idx2, z64)
  (p2,) = _mlp2(h1, agg2, batchp, W3, r(b3), r(g2), r(be2),
                r(m2), r(v2), W4, r(b4))
  return _final(p1, p2, Wl, bl.reshape(1, C))


def kernel(x, edge_index, batch, W1, b1, g1, be1, m1, v1, W2, b2, W3, b3, g2,
           be2, m2, v2, W4, b4, Wl, bl):
  return _impl(x, edge_index, batch, W1, b1, g1, be1, m1, v1, W2, b2, W3, b3,
               g2, be2, m2, v2, W4, b4, Wl, bl)
